# baseline (speedup 1.0000x reference)
import functools

import jax
import jax.numpy as jnp
from jax import lax
from jax.experimental import pallas as pl
from jax.experimental.pallas import tpu as pltpu
from jax.experimental.pallas import tpu_sc as plsc

D = 16
CHUNK = 128
NC = 2
NS = 16
NW = NC * NS
NBUF = 6
LA = 4


def _proj_kernel(x_ref, w_ref, brow_ref, h_ref):
    h_ref[...] = jnp.dot(x_ref[...], w_ref[...],
                         preferred_element_type=jnp.float32) + brow_ref[...]


def _mlp_kernel(part_ref, w1_ref, b1_ref, w2_ref, b2_ref, o_ref):
    p = part_ref[0] + part_ref[1]
    p = jnp.where(p >= 0, p, 0.01 * p)
    p = jnp.dot(p, w1_ref[...], preferred_element_type=jnp.float32) + b1_ref[...]
    p = jnp.where(p >= 0, p, 0.01 * p)
    o_ref[...] = jnp.dot(p, w2_ref[...], preferred_element_type=jnp.float32) + b2_ref[...]


def _make_agg(n_nodes, n_edges, interpret=False):
    acc_rows = ((n_nodes + 8 * NS - 1) // (8 * NS)) * (8 * NS)
    zrows = acc_rows // NS
    last = n_nodes - (NS - 1) * zrows
    assert 0 < last <= zrows
    assert n_edges % CHUNK == 0
    nrows = n_edges // CHUNK
    base_cpt = nrows // NW
    extra = nrows - base_cpt * NW
    ngrp = base_cpt // NBUF
    rest = base_cpt - ngrp * NBUF
    assert ngrp >= 1
    mesh = plsc.VectorSubcoreMesh(core_axis_name="c", subcore_axis_name="s",
                                  num_cores=NC, num_subcores=NS)

    @functools.partial(
        pl.kernel,
        out_type=jax.ShapeDtypeStruct((NC, n_nodes, D), jnp.float32),
        mesh=mesh,
        scratch_types=[
            pltpu.VMEM((base_cpt + 1, 2, CHUNK), jnp.int32),
            pltpu.VMEM((NBUF, CHUNK, D), jnp.float32),
            pltpu.VMEM((zrows, D), jnp.float32),
            pltpu.VMEM_SHARED((acc_rows, D), jnp.float32),
            [pltpu.SemaphoreType.DMA] * NBUF,
            [pltpu.SemaphoreType.DMA] * NBUF,
        ],
        compiler_params=pltpu.CompilerParams(use_tc_tiling_on_sc=False),
        interpret=interpret,
    )
    def agg(y_hbm, ei_hbm, out_hbm, ei_v, rows_v, zero_v, acc_sh,
            gsems, ssems):
        c = lax.axis_index("c")
        s = lax.axis_index("s")
        wid = s * NC + c
        start = wid * base_cpt + jnp.minimum(wid, extra)

        @pl.when(c == 0)
        def _():
            @pl.when(s < NS - 1)
            def _():
                pltpu.sync_copy(
                    y_hbm.at[pl.ds(n_nodes + s * zrows, zrows)],
                    acc_sh.at[pl.ds(s * zrows, zrows)])

            @pl.when(s == NS - 1)
            def _():
                pltpu.sync_copy(
                    y_hbm.at[pl.ds(n_nodes + (NS - 1) * zrows, last)],
                    acc_sh.at[pl.ds((NS - 1) * zrows, last)])

        @pl.when(c == 1)
        def _():
            def zbody(i, carry):
                zero_v[i, :] = jnp.zeros((D,), jnp.float32)
                return carry

            lax.fori_loop(0, zrows, zbody, 0)
            pltpu.sync_copy(zero_v, acc_sh.at[pl.ds(s * zrows, zrows)])

        @pl.when(wid < extra)
        def _():
            pltpu.sync_copy(ei_hbm.at[pl.ds(start, base_cpt + 1)], ei_v)

        @pl.when(wid >= extra)
        def _():
            pltpu.sync_copy(ei_hbm.at[pl.ds(start, base_cpt)],
                            ei_v.at[pl.ds(0, base_cpt)])

        plsc.subcore_barrier()

        for b in range(LA):
            pltpu.async_copy(y_hbm.at[ei_v.at[b, 0]], rows_v.at[b], gsems[b])

        def step(j, b, bf):
            pltpu.make_async_copy(y_hbm.at[ei_v.at[j, 0]], rows_v.at[b],
                                  gsems[b]).wait()
            pltpu.async_copy(rows_v.at[b], acc_sh.at[ei_v.at[j, 1]],
                             ssems[b], add=True)
            f = j + LA

            @pl.when(f < base_cpt)
            def _():
                @pl.when(f >= NBUF)
                def _():
                    pltpu.make_async_copy(
                        rows_v.at[bf], acc_sh.at[ei_v.at[f - NBUF, 1]],
                        ssems[bf]).wait()

                pltpu.async_copy(y_hbm.at[ei_v.at[f, 0]], rows_v.at[bf],
                                 gsems[bf])

        def body(g, carry):
            base = g * NBUF
            for b in range(NBUF):
                step(base + b, b, (b + LA) % NBUF)
            return carry

        lax.fori_loop(0, ngrp, body, 0)
        for j in range(ngrp * NBUF, base_cpt):
            step(j, j % NBUF, (j + LA) % NBUF)
        for j in range(base_cpt - NBUF, base_cpt):
            b = j % NBUF
            pltpu.make_async_copy(rows_v.at[b], acc_sh.at[ei_v.at[j, 1]],
                                  ssems[b]).wait()

        @pl.when(wid < extra)
        def _():
            pltpu.sync_copy(y_hbm.at[ei_v.at[base_cpt, 0]], rows_v.at[0])
            pltpu.sync_copy(rows_v.at[0], acc_sh.at[ei_v.at[base_cpt, 1]],
                            add=True)

        plsc.subcore_barrier()

        @pl.when(s < NS - 1)
        def _():
            pltpu.sync_copy(acc_sh.at[pl.ds(s * zrows, zrows)],
                            out_hbm.at[c, pl.ds(s * zrows, zrows)])

        @pl.when(s == NS - 1)
        def _():
            pltpu.sync_copy(acc_sh.at[pl.ds((NS - 1) * zrows, last)],
                            out_hbm.at[c, pl.ds((NS - 1) * zrows, last)])

    return agg


def _run(x, edge_index, W_l, b_l, W_r, W1, b1, W2, b2, interpret=False):
    n_nodes, d_in = x.shape
    n_edges = edge_index.shape[1]
    br = n_nodes

    ei3 = jnp.transpose(
        edge_index.astype(jnp.int32).reshape(2, n_edges // CHUNK, CHUNK),
        (1, 0, 2))

    w_cat = jnp.concatenate([W_l, W_r], axis=0).T
    brow = jnp.concatenate([jnp.zeros_like(b_l), b_l]).reshape(1, 2 * D)
    grid = n_nodes // br
    h = pl.pallas_call(
        _proj_kernel,
        grid=(grid,),
        in_specs=[
            pl.BlockSpec((br, d_in), lambda i: (i, 0)),
            pl.BlockSpec((d_in, 2 * D), lambda i: (0, 0)),
            pl.BlockSpec((1, 2 * D), lambda i: (0, 0)),
        ],
        out_specs=pl.BlockSpec((br, 2 * D), lambda i: (i, 0)),
        out_shape=jax.ShapeDtypeStruct((n_nodes, 2 * D), jnp.float32),
        interpret=interpret,
    )(x, w_cat, brow)

    hl = jnp.concatenate([h[:, :D], h[:, D:]], axis=0)
    part = _make_agg(n_nodes, n_edges, interpret=interpret)(hl, ei3)

    np8 = n_nodes * D // 128
    eye8 = jnp.eye(8, dtype=jnp.float32)
    out_p = pl.pallas_call(
        _mlp_kernel,
        grid=(grid,),
        in_specs=[
            pl.BlockSpec((NC, np8, 128), lambda i: (0, i, 0)),
            pl.BlockSpec((128, 128), lambda i: (0, 0)),
            pl.BlockSpec((1, 128), lambda i: (0, 0)),
            pl.BlockSpec((128, 128), lambda i: (0, 0)),
            pl.BlockSpec((1, 128), lambda i: (0, 0)),
        ],
        out_specs=pl.BlockSpec((np8, 128), lambda i: (i, 0)),
        out_shape=jax.ShapeDtypeStruct((np8, 128), jnp.float32),
        interpret=interpret,
    )(part.reshape(NC, np8, 128), jnp.kron(eye8, W1.T),
      jnp.tile(b1, 8).reshape(1, 128), jnp.kron(eye8, W2.T),
      jnp.tile(b2, 8).reshape(1, 128))
    return out_p.reshape(n_nodes, D)


def kernel(x, edge_index, W_l, b_l, W_r, W1, b1, W2, b2):
    return _run(x, edge_index, W_l, b_l, W_r, W1, b1, W2, b2)


# speedup vs baseline: 1.1564x; 1.1564x over previous
import functools

import jax
import jax.numpy as jnp
from jax import lax
from jax.experimental import pallas as pl
from jax.experimental.pallas import tpu as pltpu
from jax.experimental.pallas import tpu_sc as plsc

D = 16
CHUNK = 128
NC = 2
NS = 16
NW = NC * NS
NBUF = 6
LA = 4


def _proj_kernel(x_ref, w_ref, brow_ref, y_ref, r_ref):
    h = jnp.dot(x_ref[...], w_ref[...],
                preferred_element_type=jnp.float32) + brow_ref[...]
    y_ref[...] = h[:, :D]
    r_ref[...] = h[:, D:]


def _mlp_kernel(part_ref, w1_ref, b1_ref, w2_ref, b2_ref, o_ref):
    p = part_ref[0] + part_ref[1]
    p = jnp.where(p >= 0, p, 0.01 * p)
    p = jnp.dot(p, w1_ref[...], preferred_element_type=jnp.float32) + b1_ref[...]
    p = jnp.where(p >= 0, p, 0.01 * p)
    o_ref[...] = jnp.dot(p, w2_ref[...], preferred_element_type=jnp.float32) + b2_ref[...]


def _make_agg(n_nodes, n_edges, interpret=False):
    acc_rows = ((n_nodes + 8 * NS - 1) // (8 * NS)) * (8 * NS)
    zrows = acc_rows // NS
    last = n_nodes - (NS - 1) * zrows
    assert 0 < last <= zrows
    assert n_edges % CHUNK == 0
    nrows = n_edges // CHUNK
    base_cpt = nrows // NW
    extra = nrows - base_cpt * NW
    ngrp = base_cpt // NBUF
    rest = base_cpt - ngrp * NBUF
    assert ngrp >= 1
    mesh = plsc.VectorSubcoreMesh(core_axis_name="c", subcore_axis_name="s",
                                  num_cores=NC, num_subcores=NS)

    @functools.partial(
        pl.kernel,
        out_type=jax.ShapeDtypeStruct((NC, n_nodes, D), jnp.float32),
        mesh=mesh,
        scratch_types=[
            pltpu.VMEM((base_cpt + 1, 2, CHUNK), jnp.int32),
            pltpu.VMEM((NBUF, CHUNK, D), jnp.float32),
            pltpu.VMEM((zrows, D), jnp.float32),
            pltpu.VMEM_SHARED((acc_rows, D), jnp.float32),
            [pltpu.SemaphoreType.DMA] * NBUF,
            [pltpu.SemaphoreType.DMA] * NBUF,
        ],
        compiler_params=pltpu.CompilerParams(use_tc_tiling_on_sc=False),
        interpret=interpret,
    )
    def agg(y_hbm, r_hbm, ei_hbm, out_hbm, ei_v, rows_v, zero_v, acc_sh,
            gsems, ssems):
        c = lax.axis_index("c")
        s = lax.axis_index("s")
        wid = s * NC + c
        start = wid * base_cpt + jnp.minimum(wid, extra)

        @pl.when(c == 0)
        def _():
            @pl.when(s < NS - 1)
            def _():
                pltpu.sync_copy(r_hbm.at[pl.ds(s * zrows, zrows)],
                                acc_sh.at[pl.ds(s * zrows, zrows)])

            @pl.when(s == NS - 1)
            def _():
                pltpu.sync_copy(r_hbm.at[pl.ds((NS - 1) * zrows, last)],
                                acc_sh.at[pl.ds((NS - 1) * zrows, last)])

        @pl.when(c == 1)
        def _():
            def zbody(i, carry):
                zero_v[i, :] = jnp.zeros((D,), jnp.float32)
                return carry

            lax.fori_loop(0, zrows, zbody, 0)
            pltpu.sync_copy(zero_v, acc_sh.at[pl.ds(s * zrows, zrows)])

        @pl.when(wid < extra)
        def _():
            pltpu.sync_copy(ei_hbm.at[pl.ds(start, base_cpt + 1)], ei_v)

        @pl.when(wid >= extra)
        def _():
            pltpu.sync_copy(ei_hbm.at[pl.ds(start, base_cpt)],
                            ei_v.at[pl.ds(0, base_cpt)])

        plsc.subcore_barrier()

        for b in range(LA):
            pltpu.async_copy(y_hbm.at[ei_v.at[b, 0]], rows_v.at[b], gsems[b])

        def step(j, b, bf):
            pltpu.make_async_copy(y_hbm.at[ei_v.at[j, 0]], rows_v.at[b],
                                  gsems[b]).wait()
            pltpu.async_copy(rows_v.at[b], acc_sh.at[ei_v.at[j, 1]],
                             ssems[b], add=True)
            f = j + LA

            @pl.when(f < base_cpt)
            def _():
                @pl.when(f >= NBUF)
                def _():
                    pltpu.make_async_copy(
                        rows_v.at[bf], acc_sh.at[ei_v.at[f - NBUF, 1]],
                        ssems[bf]).wait()

                pltpu.async_copy(y_hbm.at[ei_v.at[f, 0]], rows_v.at[bf],
                                 gsems[bf])

        def body(g, carry):
            base = g * NBUF
            for b in range(NBUF):
                step(base + b, b, (b + LA) % NBUF)
            return carry

        lax.fori_loop(0, ngrp, body, 0)
        for j in range(ngrp * NBUF, base_cpt):
            step(j, j % NBUF, (j + LA) % NBUF)
        for j in range(base_cpt - NBUF, base_cpt):
            b = j % NBUF
            pltpu.make_async_copy(rows_v.at[b], acc_sh.at[ei_v.at[j, 1]],
                                  ssems[b]).wait()

        @pl.when(wid < extra)
        def _():
            pltpu.sync_copy(y_hbm.at[ei_v.at[base_cpt, 0]], rows_v.at[0])
            pltpu.sync_copy(rows_v.at[0], acc_sh.at[ei_v.at[base_cpt, 1]],
                            add=True)

        plsc.subcore_barrier()

        @pl.when(s < NS - 1)
        def _():
            pltpu.sync_copy(acc_sh.at[pl.ds(s * zrows, zrows)],
                            out_hbm.at[c, pl.ds(s * zrows, zrows)])

        @pl.when(s == NS - 1)
        def _():
            pltpu.sync_copy(acc_sh.at[pl.ds((NS - 1) * zrows, last)],
                            out_hbm.at[c, pl.ds((NS - 1) * zrows, last)])

    return agg


def _run(x, edge_index, W_l, b_l, W_r, W1, b1, W2, b2, interpret=False):
    n_nodes, d_in = x.shape
    n_edges = edge_index.shape[1]
    br = n_nodes

    ei3 = jnp.transpose(
        edge_index.astype(jnp.int32).reshape(2, n_edges // CHUNK, CHUNK),
        (1, 0, 2))

    w_cat = jnp.concatenate([W_l, W_r], axis=0).T
    brow = jnp.concatenate([jnp.zeros_like(b_l), b_l]).reshape(1, 2 * D)
    grid = n_nodes // br
    h = pl.pallas_call(
        _proj_kernel,
        grid=(grid,),
        in_specs=[
            pl.BlockSpec((br, d_in), lambda i: (i, 0)),
            pl.BlockSpec((d_in, 2 * D), lambda i: (0, 0)),
            pl.BlockSpec((1, 2 * D), lambda i: (0, 0)),
        ],
        out_specs=[
            pl.BlockSpec((br, D), lambda i: (i, 0)),
            pl.BlockSpec((br, D), lambda i: (i, 0)),
        ],
        out_shape=[jax.ShapeDtypeStruct((n_nodes, D), jnp.float32)] * 2,
        interpret=interpret,
    )(x, w_cat, brow)

    part = _make_agg(n_nodes, n_edges, interpret=interpret)(h[0], h[1], ei3)

    np8 = n_nodes * D // 128
    eye8 = jnp.eye(8, dtype=jnp.float32)
    out_p = pl.pallas_call(
        _mlp_kernel,
        grid=(grid,),
        in_specs=[
            pl.BlockSpec((NC, np8, 128), lambda i: (0, i, 0)),
            pl.BlockSpec((128, 128), lambda i: (0, 0)),
            pl.BlockSpec((1, 128), lambda i: (0, 0)),
            pl.BlockSpec((128, 128), lambda i: (0, 0)),
            pl.BlockSpec((1, 128), lambda i: (0, 0)),
        ],
        out_specs=pl.BlockSpec((np8, 128), lambda i: (i, 0)),
        out_shape=jax.ShapeDtypeStruct((np8, 128), jnp.float32),
        interpret=interpret,
    )(part.reshape(NC, np8, 128), jnp.kron(eye8, W1.T),
      jnp.tile(b1, 8).reshape(1, 128), jnp.kron(eye8, W2.T),
      jnp.tile(b2, 8).reshape(1, 128))
    return out_p.reshape(n_nodes, D)


def kernel(x, edge_index, W_l, b_l, W_r, W1, b1, W2, b2):
    return _run(x, edge_index, W_l, b_l, W_r, W1, b1, W2, b2)


# speedup vs baseline: 1.1652x; 1.0076x over previous
import functools

import jax
import jax.numpy as jnp
from jax import lax
from jax.experimental import pallas as pl
from jax.experimental.pallas import tpu as pltpu
from jax.experimental.pallas import tpu_sc as plsc

D = 16
CHUNK = 128
NC = 2
NS = 16
NW = NC * NS
NBUF = 6
LA = 4


def _proj_kernel(x_ref, w_ref, brow_ref, y_ref, r_ref):
    h = jnp.dot(x_ref[...], w_ref[...],
                preferred_element_type=jnp.float32) + brow_ref[...]
    y_ref[...] = h[:, :D]
    r_ref[...] = h[:, D:]


def _mlp_kernel(part_ref, w1_ref, b1_ref, w2_ref, b2_ref, o_ref):
    p = part_ref[0] + part_ref[1]
    p = jnp.where(p >= 0, p, 0.01 * p)
    p = jnp.dot(p, w1_ref[...], preferred_element_type=jnp.float32) + b1_ref[...]
    p = jnp.where(p >= 0, p, 0.01 * p)
    o_ref[...] = jnp.dot(p, w2_ref[...], preferred_element_type=jnp.float32) + b2_ref[...]


def _make_agg(n_nodes, n_edges, interpret=False):
    acc_rows = ((n_nodes + 8 * NS - 1) // (8 * NS)) * (8 * NS)
    zrows = acc_rows // NS
    last = n_nodes - (NS - 1) * zrows
    assert 0 < last <= zrows
    assert n_edges % CHUNK == 0
    nrows = n_edges // CHUNK
    base_cpt = nrows // NW
    extra = nrows - base_cpt * NW
    ngrp = base_cpt // NBUF
    rest = base_cpt - ngrp * NBUF
    assert ngrp >= 1
    mesh = plsc.VectorSubcoreMesh(core_axis_name="c", subcore_axis_name="s",
                                  num_cores=NC, num_subcores=NS)

    @functools.partial(
        pl.kernel,
        out_type=jax.ShapeDtypeStruct((NC, n_nodes, D), jnp.float32),
        mesh=mesh,
        scratch_types=[
            pltpu.VMEM((base_cpt + 1, 2, CHUNK), jnp.int32),
            pltpu.VMEM((NBUF, CHUNK, D), jnp.float32),
            pltpu.VMEM_SHARED((acc_rows, D), jnp.float32),
            [pltpu.SemaphoreType.DMA] * NBUF,
            [pltpu.SemaphoreType.DMA] * NBUF,
        ],
        compiler_params=pltpu.CompilerParams(use_tc_tiling_on_sc=False),
        interpret=interpret,
    )
    def agg(y_hbm, r_hbm, ei_hbm, z_hbm, out_hbm, ei_v, rows_v, acc_sh,
            gsems, ssems):
        c = lax.axis_index("c")
        s = lax.axis_index("s")
        wid = s * NC + c
        start = wid * base_cpt + jnp.minimum(wid, extra)

        @pl.when(c == 0)
        def _():
            @pl.when(s < NS - 1)
            def _():
                pltpu.sync_copy(r_hbm.at[pl.ds(s * zrows, zrows)],
                                acc_sh.at[pl.ds(s * zrows, zrows)])

            @pl.when(s == NS - 1)
            def _():
                pltpu.sync_copy(r_hbm.at[pl.ds((NS - 1) * zrows, last)],
                                acc_sh.at[pl.ds((NS - 1) * zrows, last)])

        @pl.when(c == 1)
        def _():
            pltpu.sync_copy(z_hbm, acc_sh.at[pl.ds(s * zrows, zrows)])

        @pl.when(wid < extra)
        def _():
            pltpu.sync_copy(ei_hbm.at[pl.ds(start, base_cpt + 1)], ei_v)

        @pl.when(wid >= extra)
        def _():
            pltpu.sync_copy(ei_hbm.at[pl.ds(start, base_cpt)],
                            ei_v.at[pl.ds(0, base_cpt)])

        plsc.subcore_barrier()

        for b in range(LA):
            pltpu.async_copy(y_hbm.at[ei_v.at[b, 0]], rows_v.at[b], gsems[b])

        def step(j, b, bf):
            pltpu.make_async_copy(y_hbm.at[ei_v.at[j, 0]], rows_v.at[b],
                                  gsems[b]).wait()
            pltpu.async_copy(rows_v.at[b], acc_sh.at[ei_v.at[j, 1]],
                             ssems[b], add=True)
            f = j + LA

            @pl.when(f < base_cpt)
            def _():
                @pl.when(f >= NBUF)
                def _():
                    pltpu.make_async_copy(
                        rows_v.at[bf], acc_sh.at[ei_v.at[f - NBUF, 1]],
                        ssems[bf]).wait()

                pltpu.async_copy(y_hbm.at[ei_v.at[f, 0]], rows_v.at[bf],
                                 gsems[bf])

        def body(g, carry):
            base = g * NBUF
            for b in range(NBUF):
                step(base + b, b, (b + LA) % NBUF)
            return carry

        lax.fori_loop(0, ngrp, body, 0)
        for j in range(ngrp * NBUF, base_cpt):
            step(j, j % NBUF, (j + LA) % NBUF)
        for j in range(base_cpt - NBUF, base_cpt):
            b = j % NBUF
            pltpu.make_async_copy(rows_v.at[b], acc_sh.at[ei_v.at[j, 1]],
                                  ssems[b]).wait()

        @pl.when(wid < extra)
        def _():
            pltpu.sync_copy(y_hbm.at[ei_v.at[base_cpt, 0]], rows_v.at[0])
            pltpu.sync_copy(rows_v.at[0], acc_sh.at[ei_v.at[base_cpt, 1]],
                            add=True)

        plsc.subcore_barrier()

        @pl.when(s < NS - 1)
        def _():
            pltpu.sync_copy(acc_sh.at[pl.ds(s * zrows, zrows)],
                            out_hbm.at[c, pl.ds(s * zrows, zrows)])

        @pl.when(s == NS - 1)
        def _():
            pltpu.sync_copy(acc_sh.at[pl.ds((NS - 1) * zrows, last)],
                            out_hbm.at[c, pl.ds((NS - 1) * zrows, last)])

    return agg


def _run(x, edge_index, W_l, b_l, W_r, W1, b1, W2, b2, interpret=False):
    n_nodes, d_in = x.shape
    n_edges = edge_index.shape[1]
    br = n_nodes

    ei3 = jnp.transpose(
        edge_index.astype(jnp.int32).reshape(2, n_edges // CHUNK, CHUNK),
        (1, 0, 2))

    w_cat = jnp.concatenate([W_l, W_r], axis=0).T
    brow = jnp.concatenate([jnp.zeros_like(b_l), b_l]).reshape(1, 2 * D)
    grid = n_nodes // br
    h = pl.pallas_call(
        _proj_kernel,
        grid=(grid,),
        in_specs=[
            pl.BlockSpec((br, d_in), lambda i: (i, 0)),
            pl.BlockSpec((d_in, 2 * D), lambda i: (0, 0)),
            pl.BlockSpec((1, 2 * D), lambda i: (0, 0)),
        ],
        out_specs=[
            pl.BlockSpec((br, D), lambda i: (i, 0)),
            pl.BlockSpec((br, D), lambda i: (i, 0)),
        ],
        out_shape=[jax.ShapeDtypeStruct((n_nodes, D), jnp.float32)] * 2,
        interpret=interpret,
    )(x, w_cat, brow)

    zrows = (((n_nodes + 8 * NS - 1) // (8 * NS)) * (8 * NS)) // NS
    part = _make_agg(n_nodes, n_edges, interpret=interpret)(
        h[0], h[1], ei3, jnp.zeros((zrows, D), jnp.float32))

    np8 = n_nodes * D // 128
    eye8 = jnp.eye(8, dtype=jnp.float32)
    out_p = pl.pallas_call(
        _mlp_kernel,
        grid=(grid,),
        in_specs=[
            pl.BlockSpec((NC, np8, 128), lambda i: (0, i, 0)),
            pl.BlockSpec((128, 128), lambda i: (0, 0)),
            pl.BlockSpec((1, 128), lambda i: (0, 0)),
            pl.BlockSpec((128, 128), lambda i: (0, 0)),
            pl.BlockSpec((1, 128), lambda i: (0, 0)),
        ],
        out_specs=pl.BlockSpec((np8, 128), lambda i: (i, 0)),
        out_shape=jax.ShapeDtypeStruct((np8, 128), jnp.float32),
        interpret=interpret,
    )(part.reshape(NC, np8, 128), jnp.kron(eye8, W1.T),
      jnp.tile(b1, 8).reshape(1, 128), jnp.kron(eye8, W2.T),
      jnp.tile(b2, 8).reshape(1, 128))
    return out_p.reshape(n_nodes, D)


def kernel(x, edge_index, W_l, b_l, W_r, W1, b1, W2, b2):
    return _run(x, edge_index, W_l, b_l, W_r, W1, b1, W2, b2)
